# Initial kernel scaffold; baseline (speedup 1.0000x reference)
import jax
import jax.numpy as jnp
from jax.experimental import pallas as pl


def kernel(x, edge_index, batch, mol_wt, n_rings, W1, b1, W2, b2, W3, b3, fc1_W, fc1_b, fc2_W, fc2_b):
    raise NotImplementedError("write your pallas kernel here")


# speedup vs baseline: 13.7079x; 13.7079x over previous
import functools

import jax
import jax.numpy as jnp
from jax import lax
from jax.experimental import pallas as pl
from jax.experimental.pallas import tpu as pltpu
from jax.experimental.pallas import tpu_sc as plsc

N = 10000
E = 320000
F_IN = 128
H = 256
HH = H // 2
G = 64

CHUNK = 125
NCHUNKS = E // CHUNK
NSUB = 16
CPT = NCHUNKS // NSUB
KGRP = 32
NGRP = CPT // KGRP

DEG_W = 16
NPAD = 10240
ROWS_PER_SUB = NPAD // NSUB
DEG_ROWS = NPAD // NSUB
DEG_CPT = NCHUNKS // 2 // NSUB

RBLK = 1000
GRID = N // RBLK


@functools.cache
def _sc_kernels():
    mesh = plsc.VectorSubcoreMesh(
        core_axis_name="c", subcore_axis_name="s", num_cores=2, num_subcores=NSUB
    )

    @functools.partial(
        pl.kernel,
        out_type=(
            jax.ShapeDtypeStruct((NPAD, DEG_W), jnp.float32),
            jax.ShapeDtypeStruct((NPAD, DEG_W), jnp.float32),
        ),
        mesh=mesh,
        scratch_types=[
            pltpu.VMEM_SHARED((NPAD, DEG_W), jnp.float32),
            pltpu.VMEM((DEG_CPT, CHUNK), jnp.int32),
            pltpu.VMEM((CHUNK, DEG_W), jnp.float32),
        ],
    )
    def deg_kernel(dst2d, ones, zeros, deg_a, deg_b, acc, idx_v, ones_v):
        c = lax.axis_index("c")
        s = lax.axis_index("s")
        rbase = s * DEG_ROWS
        pltpu.sync_copy(zeros, acc.at[pl.ds(rbase, DEG_ROWS)])
        pltpu.sync_copy(ones, ones_v)
        cbase = c * (NCHUNKS // 2) + s * DEG_CPT
        pltpu.sync_copy(dst2d.at[pl.ds(cbase, DEG_CPT)], idx_v)
        plsc.subcore_barrier()

        def body(j, carry):
            pltpu.sync_copy(ones_v, acc.at[idx_v.at[j]], add=True)
            return carry

        lax.fori_loop(0, DEG_CPT, body, 0)
        plsc.subcore_barrier()

        @pl.when(c == 0)
        def _():
            pltpu.sync_copy(acc.at[pl.ds(rbase, DEG_ROWS)],
                            deg_a.at[pl.ds(rbase, DEG_ROWS)])

        @pl.when(c == 1)
        def _():
            pltpu.sync_copy(acc.at[pl.ds(rbase, DEG_ROWS)],
                            deg_b.at[pl.ds(rbase, DEG_ROWS)])

    @functools.partial(
        pl.kernel,
        out_type=(
            jax.ShapeDtypeStruct((NPAD, HH), jnp.float32),
            jax.ShapeDtypeStruct((NPAD, HH), jnp.float32),
        ),
        mesh=mesh,
        scratch_types=[
            pltpu.VMEM_SHARED((NPAD, HH), jnp.float32),
            pltpu.VMEM((KGRP, CHUNK), jnp.int32),
            pltpu.VMEM((KGRP, CHUNK), jnp.int32),
            pltpu.VMEM((CHUNK, HH), jnp.float32),
            pltpu.SemaphoreType.DMA,
        ],
    )
    def prop_kernel(y_a, y_b, src2d, dst2d, agg_a, agg_b,
                    acc, src_v, dst_v, rows_v, sem):
        c = lax.axis_index("c")
        s = lax.axis_index("s")
        rbase = pl.multiple_of(s * ROWS_PER_SUB, 8)
        cbase = s * CPT

        @pl.when(c == 0)
        def _():
            pltpu.sync_copy(y_a.at[pl.ds(rbase, ROWS_PER_SUB)],
                            acc.at[pl.ds(rbase, ROWS_PER_SUB)])

        @pl.when(c == 1)
        def _():
            pltpu.sync_copy(y_b.at[pl.ds(rbase, ROWS_PER_SUB)],
                            acc.at[pl.ds(rbase, ROWS_PER_SUB)])

        plsc.subcore_barrier()

        def gbody(g, carry):
            gb = pl.multiple_of(cbase + g * KGRP, 8)
            pltpu.sync_copy(src2d.at[pl.ds(gb, KGRP)], src_v)
            pltpu.sync_copy(dst2d.at[pl.ds(gb, KGRP)], dst_v)

            def body(j, carry2):
                @pl.when(c == 0)
                def _():
                    pltpu.async_copy(y_a.at[src_v.at[j]], rows_v, sem).wait()

                @pl.when(c == 1)
                def _():
                    pltpu.async_copy(y_b.at[src_v.at[j]], rows_v, sem).wait()

                pltpu.sync_copy(rows_v, acc.at[dst_v.at[j]], add=True)
                return carry2

            lax.fori_loop(0, KGRP, body, 0)
            return carry

        lax.fori_loop(0, NGRP, gbody, 0)
        plsc.subcore_barrier()

        @pl.when(c == 0)
        def _():
            pltpu.sync_copy(acc.at[pl.ds(rbase, ROWS_PER_SUB)],
                            agg_a.at[pl.ds(rbase, ROWS_PER_SUB)])

        @pl.when(c == 1)
        def _():
            pltpu.sync_copy(acc.at[pl.ds(rbase, ROWS_PER_SUB)],
                            agg_b.at[pl.ds(rbase, ROWS_PER_SUB)])

    return deg_kernel, prop_kernel


def _l1_body(x_ref, w_ref, da_ref, db_ref, ya_ref, yb_ref, dinv_ref):
    deg = da_ref[:, 0:1] + db_ref[:, 0:1] + 1.0
    dinv = lax.rsqrt(deg)
    y = jnp.dot(x_ref[...], w_ref[...], preferred_element_type=jnp.float32) * dinv
    ya_ref[...] = y[:, :HH]
    yb_ref[...] = y[:, HH:]
    dinv_ref[...] = dinv


@functools.cache
def _l1_call():
    return pl.pallas_call(
        _l1_body,
        grid=(GRID,),
        in_specs=[
            pl.BlockSpec((RBLK, F_IN), lambda b: (b, 0)),
            pl.BlockSpec((F_IN, H), lambda b: (0, 0)),
            pl.BlockSpec((RBLK, DEG_W), lambda b: (b, 0)),
            pl.BlockSpec((RBLK, DEG_W), lambda b: (b, 0)),
        ],
        out_specs=[
            pl.BlockSpec((RBLK, HH), lambda b: (b, 0)),
            pl.BlockSpec((RBLK, HH), lambda b: (b, 0)),
            pl.BlockSpec((RBLK, 1), lambda b: (b, 0)),
        ],
        out_shape=[
            jax.ShapeDtypeStruct((NPAD, HH), jnp.float32),
            jax.ShapeDtypeStruct((NPAD, HH), jnp.float32),
            jax.ShapeDtypeStruct((N, 1), jnp.float32),
        ],
    )


def _mid_body(aa_ref, ab_ref, dinv_ref, ba_ref, bb_ref, wa_ref, wb_ref,
              ya_ref, yb_ref):
    dinv = dinv_ref[...]
    ha = jnp.maximum(aa_ref[...] * dinv + ba_ref[...], 0.0)
    hb = jnp.maximum(ab_ref[...] * dinv + bb_ref[...], 0.0)
    y = (jnp.dot(ha, wa_ref[...], preferred_element_type=jnp.float32)
         + jnp.dot(hb, wb_ref[...], preferred_element_type=jnp.float32)) * dinv
    ya_ref[...] = y[:, :HH]
    yb_ref[...] = y[:, HH:]


@functools.cache
def _mid_call():
    return pl.pallas_call(
        _mid_body,
        grid=(GRID,),
        in_specs=[
            pl.BlockSpec((RBLK, HH), lambda b: (b, 0)),
            pl.BlockSpec((RBLK, HH), lambda b: (b, 0)),
            pl.BlockSpec((RBLK, 1), lambda b: (b, 0)),
            pl.BlockSpec((1, HH), lambda b: (0, 0)),
            pl.BlockSpec((1, HH), lambda b: (0, 0)),
            pl.BlockSpec((HH, H), lambda b: (0, 0)),
            pl.BlockSpec((HH, H), lambda b: (0, 0)),
        ],
        out_specs=[
            pl.BlockSpec((RBLK, HH), lambda b: (b, 0)),
            pl.BlockSpec((RBLK, HH), lambda b: (b, 0)),
        ],
        out_shape=[
            jax.ShapeDtypeStruct((NPAD, HH), jnp.float32),
            jax.ShapeDtypeStruct((NPAD, HH), jnp.float32),
        ],
    )


def _fin_body(aa_ref, ab_ref, dinv_ref, ba_ref, bb_ref, batch_ref, ext_ref,
              f1m_ref, f1e_ref, f1b_ref, f2w_ref, f2b_ref, out_ref,
              sums, counts):
    b = pl.program_id(0)

    @pl.when(b == 0)
    def _():
        sums[...] = jnp.zeros_like(sums)
        counts[...] = jnp.zeros_like(counts)

    dinv = dinv_ref[...]
    ha = jnp.maximum(aa_ref[...] * dinv + ba_ref[...], 0.0)
    hb = jnp.maximum(ab_ref[...] * dinv + bb_ref[...], 0.0)
    h = jnp.concatenate([ha, hb], axis=1)
    gi = lax.broadcasted_iota(jnp.int32, (1, G), 1)
    m = (batch_ref[...] == gi).astype(jnp.float32)
    sums[...] += lax.dot_general(m, h, (((0,), (0,)), ((), ())),
                                 preferred_element_type=jnp.float32)
    counts[...] += jnp.sum(m, axis=0)[:, None]

    @pl.when(b == GRID - 1)
    def _():
        hg = sums[...] / jnp.maximum(counts[...], 1.0)
        z = (jnp.dot(hg, f1m_ref[...], preferred_element_type=jnp.float32)
             + jnp.dot(ext_ref[...], f1e_ref[...],
                       preferred_element_type=jnp.float32)
             + f1b_ref[...])
        z = jnp.maximum(z, 0.0)
        out_ref[...] = (jnp.dot(z, f2w_ref[...],
                                preferred_element_type=jnp.float32)
                        + f2b_ref[...])


@functools.cache
def _fin_call():
    return pl.pallas_call(
        _fin_body,
        grid=(GRID,),
        in_specs=[
            pl.BlockSpec((RBLK, HH), lambda b: (b, 0)),
            pl.BlockSpec((RBLK, HH), lambda b: (b, 0)),
            pl.BlockSpec((RBLK, 1), lambda b: (b, 0)),
            pl.BlockSpec((1, HH), lambda b: (0, 0)),
            pl.BlockSpec((1, HH), lambda b: (0, 0)),
            pl.BlockSpec((RBLK, 1), lambda b: (b, 0)),
            pl.BlockSpec((G, 2), lambda b: (0, 0)),
            pl.BlockSpec((H, 196), lambda b: (0, 0)),
            pl.BlockSpec((2, 196), lambda b: (0, 0)),
            pl.BlockSpec((1, 196), lambda b: (0, 0)),
            pl.BlockSpec((196, 1), lambda b: (0, 0)),
            pl.BlockSpec((1, 1), lambda b: (0, 0)),
        ],
        out_specs=pl.BlockSpec((G, 1), lambda b: (0, 0)),
        out_shape=jax.ShapeDtypeStruct((G, 1), jnp.float32),
        scratch_shapes=[
            pltpu.VMEM((G, H), jnp.float32),
            pltpu.VMEM((G, 1), jnp.float32),
        ],
    )


def kernel(x, edge_index, batch, mol_wt, n_rings, W1, b1, W2, b2, W3, b3,
           fc1_W, fc1_b, fc2_W, fc2_b):
    deg_kernel, prop_kernel = _sc_kernels()
    src2d = edge_index[0].reshape(NCHUNKS, CHUNK)
    dst2d = edge_index[1].reshape(NCHUNKS, CHUNK)
    ones = jnp.ones((CHUNK, DEG_W), jnp.float32)
    zeros = jnp.zeros((DEG_ROWS, DEG_W), jnp.float32)

    deg_a, deg_b = deg_kernel(dst2d, ones, zeros)
    ya, yb, dinv = _l1_call()(x, W1, deg_a, deg_b)
    agg_a, agg_b = prop_kernel(ya, yb, src2d, dst2d)
    ya, yb = _mid_call()(agg_a, agg_b, dinv,
                         b1[:HH].reshape(1, HH), b1[HH:].reshape(1, HH),
                         W2[:HH], W2[HH:])
    agg_a, agg_b = prop_kernel(ya, yb, src2d, dst2d)
    ya, yb = _mid_call()(agg_a, agg_b, dinv,
                         b2[:HH].reshape(1, HH), b2[HH:].reshape(1, HH),
                         W3[:HH], W3[HH:])
    agg_a, agg_b = prop_kernel(ya, yb, src2d, dst2d)

    ext = jnp.concatenate([mol_wt, n_rings], axis=1)
    out = _fin_call()(agg_a, agg_b, dinv,
                      b3[:HH].reshape(1, HH), b3[HH:].reshape(1, HH),
                      batch.reshape(N, 1), ext,
                      fc1_W[:H], fc1_W[H:], fc1_b.reshape(1, 196),
                      fc2_W, fc2_b.reshape(1, 1))
    return out


# speedup vs baseline: 18.6046x; 1.3572x over previous
import functools

import jax
import jax.numpy as jnp
from jax import lax
from jax.experimental import pallas as pl
from jax.experimental.pallas import tpu as pltpu
from jax.experimental.pallas import tpu_sc as plsc

N = 10000
E = 320000
F_IN = 128
H = 256
HH = H // 2
G = 64

CHUNK = 125
NCHUNKS = E // CHUNK
NSUB = 16
CPT = NCHUNKS // NSUB
KGRP = 16
NGRP = CPT // KGRP

DEG_W = 16
NPAD = 10240
ROWS_PER_SUB = NPAD // NSUB
DEG_ROWS = NPAD // NSUB
DEG_CPT = NCHUNKS // 2 // NSUB

RBLK = 1000
GRID = N // RBLK


@functools.cache
def _sc_kernels():
    mesh = plsc.VectorSubcoreMesh(
        core_axis_name="c", subcore_axis_name="s", num_cores=2, num_subcores=NSUB
    )

    @functools.partial(
        pl.kernel,
        out_type=(
            jax.ShapeDtypeStruct((NPAD, DEG_W), jnp.float32),
            jax.ShapeDtypeStruct((NPAD, DEG_W), jnp.float32),
        ),
        mesh=mesh,
        scratch_types=[
            pltpu.VMEM_SHARED((NPAD, DEG_W), jnp.float32),
            pltpu.VMEM((DEG_CPT, CHUNK), jnp.int32),
            pltpu.VMEM((CHUNK, DEG_W), jnp.float32),
        ],
    )
    def deg_kernel(dst2d, ones, zeros, deg_a, deg_b, acc, idx_v, ones_v):
        c = lax.axis_index("c")
        s = lax.axis_index("s")
        rbase = s * DEG_ROWS
        pltpu.sync_copy(zeros, acc.at[pl.ds(rbase, DEG_ROWS)])
        pltpu.sync_copy(ones, ones_v)
        cbase = c * (NCHUNKS // 2) + s * DEG_CPT
        pltpu.sync_copy(dst2d.at[pl.ds(cbase, DEG_CPT)], idx_v)
        plsc.subcore_barrier()

        def body(j, carry):
            pltpu.sync_copy(ones_v, acc.at[idx_v.at[j]], add=True)
            return carry

        lax.fori_loop(0, DEG_CPT, body, 0)
        plsc.subcore_barrier()

        @pl.when(c == 0)
        def _():
            pltpu.sync_copy(acc.at[pl.ds(rbase, DEG_ROWS)],
                            deg_a.at[pl.ds(rbase, DEG_ROWS)])

        @pl.when(c == 1)
        def _():
            pltpu.sync_copy(acc.at[pl.ds(rbase, DEG_ROWS)],
                            deg_b.at[pl.ds(rbase, DEG_ROWS)])

    @functools.partial(
        pl.kernel,
        out_type=(
            jax.ShapeDtypeStruct((NPAD, HH), jnp.float32),
            jax.ShapeDtypeStruct((NPAD, HH), jnp.float32),
        ),
        mesh=mesh,
        scratch_types=[
            pltpu.VMEM_SHARED((NPAD, HH), jnp.float32),
            pltpu.VMEM((2 * KGRP, CHUNK), jnp.int32),
            pltpu.VMEM((2 * KGRP, CHUNK), jnp.int32),
            pltpu.VMEM((CHUNK, HH), jnp.float32),
            pltpu.VMEM((CHUNK, HH), jnp.float32),
            pltpu.SemaphoreType.DMA,
            pltpu.SemaphoreType.DMA,
            pltpu.SemaphoreType.DMA,
            pltpu.SemaphoreType.DMA,
            pltpu.SemaphoreType.DMA,
        ],
    )
    def prop_kernel(y_a, y_b, src2d, dst2d, agg_a, agg_b,
                    acc, src_v, dst_v, rows0, rows1,
                    gsem0, gsem1, ssem0, ssem1, isem):
        c = lax.axis_index("c")
        s = lax.axis_index("s")
        rbase = pl.multiple_of(s * ROWS_PER_SUB, 8)
        cbase = s * CPT

        def idx_row(buf, j):
            return buf.at[((j // KGRP) % 2) * KGRP + j % KGRP]

        def issue_refill(g):
            gb = pl.multiple_of(cbase + g * KGRP, 8)
            vb = (g % 2) * KGRP
            pltpu.async_copy(src2d.at[pl.ds(gb, KGRP)],
                             src_v.at[pl.ds(vb, KGRP)], isem)
            pltpu.async_copy(dst2d.at[pl.ds(gb, KGRP)],
                             dst_v.at[pl.ds(vb, KGRP)], isem)

        def wait_refill(g):
            vb = (g % 2) * KGRP
            pltpu.make_async_copy(src2d.at[pl.ds(cbase, KGRP)],
                                  src_v.at[pl.ds(vb, KGRP)], isem).wait()
            pltpu.make_async_copy(dst2d.at[pl.ds(cbase, KGRP)],
                                  dst_v.at[pl.ds(vb, KGRP)], isem).wait()

        def issue_gather(j, rows, gsem):
            @pl.when(c == 0)
            def _():
                pltpu.async_copy(y_a.at[idx_row(src_v, j)], rows, gsem)

            @pl.when(c == 1)
            def _():
                pltpu.async_copy(y_b.at[idx_row(src_v, j)], rows, gsem)

        def wait_gather(j, rows, gsem):
            pltpu.make_async_copy(y_a.at[idx_row(src_v, j)], rows, gsem).wait()

        def issue_scatter(j, rows, ssem):
            pltpu.async_copy(rows, acc.at[idx_row(dst_v, j)], ssem, add=True)

        def wait_scatter(j, rows, ssem):
            pltpu.make_async_copy(rows, acc.at[idx_row(dst_v, j)], ssem).wait()

        @pl.when(c == 0)
        def _():
            pltpu.sync_copy(y_a.at[pl.ds(rbase, ROWS_PER_SUB)],
                            acc.at[pl.ds(rbase, ROWS_PER_SUB)])

        @pl.when(c == 1)
        def _():
            pltpu.sync_copy(y_b.at[pl.ds(rbase, ROWS_PER_SUB)],
                            acc.at[pl.ds(rbase, ROWS_PER_SUB)])

        plsc.subcore_barrier()

        issue_refill(0)
        wait_refill(0)
        issue_gather(0, rows0, gsem0)

        def pair(q, carry):
            for k in (0, 1):
                j = 2 * q + k
                rows_k, gsem_k, ssem_k = (
                    (rows0, gsem0, ssem0) if k == 0 else (rows1, gsem1, ssem1))
                rows_o, gsem_o, ssem_o = (
                    (rows1, gsem1, ssem1) if k == 0 else (rows0, gsem0, ssem0))
                boundary = j % KGRP == 0
                g = j // KGRP

                @pl.when(jnp.logical_and(j >= 1, boundary))
                def _():
                    wait_scatter(j - 1, rows_o, ssem_o)

                @pl.when(jnp.logical_and(boundary, j + KGRP < CPT))
                def _():
                    issue_refill(g + 1)

                wait_gather(j, rows_k, gsem_k)
                issue_scatter(j, rows_k, ssem_k)

                @pl.when(jnp.logical_and(j >= 1, jnp.logical_not(boundary)))
                def _():
                    wait_scatter(j - 1, rows_o, ssem_o)

                @pl.when(j + 1 < CPT)
                def _():
                    @pl.when((j + 1) % KGRP == 0)
                    def _():
                        wait_refill((j + 1) // KGRP)

                    issue_gather(j + 1, rows_o, gsem_o)
            return carry

        lax.fori_loop(0, CPT // 2, pair, 0)
        wait_scatter(CPT - 1, rows1, ssem1)
        plsc.subcore_barrier()

        @pl.when(c == 0)
        def _():
            pltpu.sync_copy(acc.at[pl.ds(rbase, ROWS_PER_SUB)],
                            agg_a.at[pl.ds(rbase, ROWS_PER_SUB)])

        @pl.when(c == 1)
        def _():
            pltpu.sync_copy(acc.at[pl.ds(rbase, ROWS_PER_SUB)],
                            agg_b.at[pl.ds(rbase, ROWS_PER_SUB)])

    return deg_kernel, prop_kernel


def _l1_body(x_ref, w_ref, da_ref, db_ref, ya_ref, yb_ref, dinv_ref):
    deg = da_ref[:, 0:1] + db_ref[:, 0:1] + 1.0
    dinv = lax.rsqrt(deg)
    y = jnp.dot(x_ref[...], w_ref[...], preferred_element_type=jnp.float32) * dinv
    ya_ref[...] = y[:, :HH]
    yb_ref[...] = y[:, HH:]
    dinv_ref[...] = dinv


@functools.cache
def _l1_call():
    return pl.pallas_call(
        _l1_body,
        grid=(GRID,),
        in_specs=[
            pl.BlockSpec((RBLK, F_IN), lambda b: (b, 0)),
            pl.BlockSpec((F_IN, H), lambda b: (0, 0)),
            pl.BlockSpec((RBLK, DEG_W), lambda b: (b, 0)),
            pl.BlockSpec((RBLK, DEG_W), lambda b: (b, 0)),
        ],
        out_specs=[
            pl.BlockSpec((RBLK, HH), lambda b: (b, 0)),
            pl.BlockSpec((RBLK, HH), lambda b: (b, 0)),
            pl.BlockSpec((RBLK, 1), lambda b: (b, 0)),
        ],
        out_shape=[
            jax.ShapeDtypeStruct((NPAD, HH), jnp.float32),
            jax.ShapeDtypeStruct((NPAD, HH), jnp.float32),
            jax.ShapeDtypeStruct((N, 1), jnp.float32),
        ],
    )


def _mid_body(aa_ref, ab_ref, dinv_ref, ba_ref, bb_ref, wa_ref, wb_ref,
              ya_ref, yb_ref):
    dinv = dinv_ref[...]
    ha = jnp.maximum(aa_ref[...] * dinv + ba_ref[...], 0.0)
    hb = jnp.maximum(ab_ref[...] * dinv + bb_ref[...], 0.0)
    y = (jnp.dot(ha, wa_ref[...], preferred_element_type=jnp.float32)
         + jnp.dot(hb, wb_ref[...], preferred_element_type=jnp.float32)) * dinv
    ya_ref[...] = y[:, :HH]
    yb_ref[...] = y[:, HH:]


@functools.cache
def _mid_call():
    return pl.pallas_call(
        _mid_body,
        grid=(GRID,),
        in_specs=[
            pl.BlockSpec((RBLK, HH), lambda b: (b, 0)),
            pl.BlockSpec((RBLK, HH), lambda b: (b, 0)),
            pl.BlockSpec((RBLK, 1), lambda b: (b, 0)),
            pl.BlockSpec((1, HH), lambda b: (0, 0)),
            pl.BlockSpec((1, HH), lambda b: (0, 0)),
            pl.BlockSpec((HH, H), lambda b: (0, 0)),
            pl.BlockSpec((HH, H), lambda b: (0, 0)),
        ],
        out_specs=[
            pl.BlockSpec((RBLK, HH), lambda b: (b, 0)),
            pl.BlockSpec((RBLK, HH), lambda b: (b, 0)),
        ],
        out_shape=[
            jax.ShapeDtypeStruct((NPAD, HH), jnp.float32),
            jax.ShapeDtypeStruct((NPAD, HH), jnp.float32),
        ],
    )


def _fin_body(aa_ref, ab_ref, dinv_ref, ba_ref, bb_ref, batch_ref, ext_ref,
              f1m_ref, f1e_ref, f1b_ref, f2w_ref, f2b_ref, out_ref,
              sums, counts):
    b = pl.program_id(0)

    @pl.when(b == 0)
    def _():
        sums[...] = jnp.zeros_like(sums)
        counts[...] = jnp.zeros_like(counts)

    dinv = dinv_ref[...]
    ha = jnp.maximum(aa_ref[...] * dinv + ba_ref[...], 0.0)
    hb = jnp.maximum(ab_ref[...] * dinv + bb_ref[...], 0.0)
    h = jnp.concatenate([ha, hb], axis=1)
    gi = lax.broadcasted_iota(jnp.int32, (1, G), 1)
    m = (batch_ref[...] == gi).astype(jnp.float32)
    sums[...] += lax.dot_general(m, h, (((0,), (0,)), ((), ())),
                                 preferred_element_type=jnp.float32)
    counts[...] += jnp.sum(m, axis=0)[:, None]

    @pl.when(b == GRID - 1)
    def _():
        hg = sums[...] / jnp.maximum(counts[...], 1.0)
        z = (jnp.dot(hg, f1m_ref[...], preferred_element_type=jnp.float32)
             + jnp.dot(ext_ref[...], f1e_ref[...],
                       preferred_element_type=jnp.float32)
             + f1b_ref[...])
        z = jnp.maximum(z, 0.0)
        out_ref[...] = (jnp.dot(z, f2w_ref[...],
                                preferred_element_type=jnp.float32)
                        + f2b_ref[...])


@functools.cache
def _fin_call():
    return pl.pallas_call(
        _fin_body,
        grid=(GRID,),
        in_specs=[
            pl.BlockSpec((RBLK, HH), lambda b: (b, 0)),
            pl.BlockSpec((RBLK, HH), lambda b: (b, 0)),
            pl.BlockSpec((RBLK, 1), lambda b: (b, 0)),
            pl.BlockSpec((1, HH), lambda b: (0, 0)),
            pl.BlockSpec((1, HH), lambda b: (0, 0)),
            pl.BlockSpec((RBLK, 1), lambda b: (b, 0)),
            pl.BlockSpec((G, 2), lambda b: (0, 0)),
            pl.BlockSpec((H, 196), lambda b: (0, 0)),
            pl.BlockSpec((2, 196), lambda b: (0, 0)),
            pl.BlockSpec((1, 196), lambda b: (0, 0)),
            pl.BlockSpec((196, 1), lambda b: (0, 0)),
            pl.BlockSpec((1, 1), lambda b: (0, 0)),
        ],
        out_specs=pl.BlockSpec((G, 1), lambda b: (0, 0)),
        out_shape=jax.ShapeDtypeStruct((G, 1), jnp.float32),
        scratch_shapes=[
            pltpu.VMEM((G, H), jnp.float32),
            pltpu.VMEM((G, 1), jnp.float32),
        ],
    )


def kernel(x, edge_index, batch, mol_wt, n_rings, W1, b1, W2, b2, W3, b3,
           fc1_W, fc1_b, fc2_W, fc2_b):
    deg_kernel, prop_kernel = _sc_kernels()
    src2d = edge_index[0].reshape(NCHUNKS, CHUNK)
    dst2d = edge_index[1].reshape(NCHUNKS, CHUNK)
    ones = jnp.ones((CHUNK, DEG_W), jnp.float32)
    zeros = jnp.zeros((DEG_ROWS, DEG_W), jnp.float32)

    deg_a, deg_b = deg_kernel(dst2d, ones, zeros)
    ya, yb, dinv = _l1_call()(x, W1, deg_a, deg_b)
    agg_a, agg_b = prop_kernel(ya, yb, src2d, dst2d)
    ya, yb = _mid_call()(agg_a, agg_b, dinv,
                         b1[:HH].reshape(1, HH), b1[HH:].reshape(1, HH),
                         W2[:HH], W2[HH:])
    agg_a, agg_b = prop_kernel(ya, yb, src2d, dst2d)
    ya, yb = _mid_call()(agg_a, agg_b, dinv,
                         b2[:HH].reshape(1, HH), b2[HH:].reshape(1, HH),
                         W3[:HH], W3[HH:])
    agg_a, agg_b = prop_kernel(ya, yb, src2d, dst2d)

    ext = jnp.concatenate([mol_wt, n_rings], axis=1)
    out = _fin_call()(agg_a, agg_b, dinv,
                      b3[:HH].reshape(1, HH), b3[HH:].reshape(1, HH),
                      batch.reshape(N, 1), ext,
                      fc1_W[:H], fc1_W[H:], fc1_b.reshape(1, 196),
                      fc2_W, fc2_b.reshape(1, 1))
    return out
